# Initial kernel scaffold; baseline (speedup 1.0000x reference)
import jax
import jax.numpy as jnp
from jax.experimental import pallas as pl


def kernel(x, edge_index, edge_index_neg, W1, b1, W2, b2, We1, be1, We2, be2):
    raise NotImplementedError("write your pallas kernel here")


# speedup vs baseline: 1.2428x; 1.2428x over previous
import functools

import jax
import jax.numpy as jnp
from jax import lax
from jax.experimental import pallas as pl
from jax.experimental.pallas import tpu as pltpu
from jax.experimental.pallas import tpu_sc as plsc

N = 10000
E = 320000
D = 128

_NC = 2
_NS = 16
_NW = _NC * _NS
_CHUNK = 80
_PER_W = E // _NW
_NCHUNK = _PER_W // _CHUNK


def _gather_mul_body(x_hbm, ea_hbm, eb_hbm, out_hbm,
                     ia_v, ib_v, ra_v, rb_v, sem_a, sem_b):
    wid = lax.axis_index("s") * _NC + lax.axis_index("c")
    w_base = wid * _PER_W

    def chunk_body(t, carry):
        base = w_base + t * _CHUNK
        pltpu.sync_copy(ea_hbm.at[pl.ds(base, _CHUNK)], ia_v)
        pltpu.sync_copy(eb_hbm.at[pl.ds(base, _CHUNK)], ib_v)
        cp_a = pltpu.async_copy(x_hbm.at[ia_v], ra_v, sem_a)
        cp_b = pltpu.async_copy(x_hbm.at[ib_v], rb_v, sem_b)
        cp_a.wait()
        cp_b.wait()

        def row_body(r, c):
            for k in range(D // 16):
                sl = pl.ds(k * 16, 16)
                ra_v[r, sl] = ra_v[r, sl] * rb_v[r, sl]
            return c

        lax.fori_loop(0, _CHUNK, row_body, 0, unroll=2)
        pltpu.sync_copy(ra_v, out_hbm.at[pl.ds(base, _CHUNK)])
        return carry

    lax.fori_loop(0, _NCHUNK, chunk_body, 0)


def _gather_mul(x, ea, eb):
    mesh = plsc.VectorSubcoreMesh(core_axis_name="c", subcore_axis_name="s")
    f = functools.partial(
        pl.kernel,
        mesh=mesh,
        out_type=jax.ShapeDtypeStruct((E, D), jnp.float32),
        scratch_types=[
            pltpu.VMEM((_CHUNK,), jnp.int32),
            pltpu.VMEM((_CHUNK,), jnp.int32),
            pltpu.VMEM((_CHUNK, D), jnp.float32),
            pltpu.VMEM((_CHUNK, D), jnp.float32),
            pltpu.SemaphoreType.DMA,
            pltpu.SemaphoreType.DMA,
        ],
    )(_gather_mul_body)
    return f(x, ea, eb)


_B = 2560
_G = E // _B


def _decode_body(ep_ref, en_ref, wcat_ref, b1_ref, be1_ref, w2t_ref, b2_ref,
                 we2_ref, be2_ref, attr_ref, pos_ref, neg_ref):
    wcat = wcat_ref[...]
    h = jnp.maximum(ep_ref[...], 0.0)
    a = jnp.dot(h, wcat)
    a1 = jnp.maximum(a[:, :D] + b1_ref[...], 0.0)
    attr_ref[...] = jax.nn.sigmoid(jnp.dot(a1, w2t_ref[...]) + b2_ref[...])
    we2 = we2_ref[...]
    be2 = be2_ref[0, 0]
    ae = jnp.maximum(a[:, D:] + be1_ref[...], 0.0)
    pos_ref[...] = jax.nn.sigmoid(jnp.sum(ae * we2, axis=1) + be2).reshape(1, 1, _B)
    hn = jnp.maximum(en_ref[...], 0.0)
    an = jnp.maximum(jnp.dot(hn, wcat[:, D:]) + be1_ref[...], 0.0)
    neg_ref[...] = jax.nn.sigmoid(jnp.sum(an * we2, axis=1) + be2).reshape(1, 1, _B)


def _decode(em_pos, em_neg, wcat_t, b1r, be1r, w2t8, b2r, we2r, be2r):
    return pl.pallas_call(
        _decode_body,
        grid=(_G,),
        in_specs=[
            pl.BlockSpec((_B, D), lambda i: (i, 0)),
            pl.BlockSpec((_B, D), lambda i: (i, 0)),
            pl.BlockSpec((D, 2 * D), lambda i: (0, 0)),
            pl.BlockSpec((1, D), lambda i: (0, 0)),
            pl.BlockSpec((1, D), lambda i: (0, 0)),
            pl.BlockSpec((D, 8), lambda i: (0, 0)),
            pl.BlockSpec((1, 8), lambda i: (0, 0)),
            pl.BlockSpec((1, D), lambda i: (0, 0)),
            pl.BlockSpec((1, 1), lambda i: (0, 0)),
        ],
        out_specs=[
            pl.BlockSpec((_B, 8), lambda i: (i, 0)),
            pl.BlockSpec((1, 1, _B), lambda i: (i, 0, 0)),
            pl.BlockSpec((1, 1, _B), lambda i: (i, 0, 0)),
        ],
        out_shape=[
            jax.ShapeDtypeStruct((E, 8), jnp.float32),
            jax.ShapeDtypeStruct((_G, 1, _B), jnp.float32),
            jax.ShapeDtypeStruct((_G, 1, _B), jnp.float32),
        ],
        compiler_params=pltpu.CompilerParams(
            dimension_semantics=("arbitrary",),
        ),
    )(em_pos, em_neg, wcat_t, b1r, be1r, w2t8, b2r, we2r, be2r)


def kernel(x, edge_index, edge_index_neg, W1, b1, W2, b2, We1, be1, We2, be2):
    em_pos = _gather_mul(x, edge_index[0], edge_index[1])
    em_neg = _gather_mul(x, edge_index_neg[0], edge_index_neg[1])

    wcat_t = jnp.concatenate([W1, We1], axis=0).T
    w2t8 = jnp.pad(W2, ((0, 1), (0, 0))).T
    b2r = jnp.pad(b2, (0, 1)).reshape(1, 8)
    attr8, pos, neg = _decode(
        em_pos, em_neg, wcat_t, b1.reshape(1, D), be1.reshape(1, D),
        w2t8, b2r, We2.reshape(1, D), be2.reshape(1, 1))
    return attr8[:, :7], pos.reshape(E), neg.reshape(E)


# speedup vs baseline: 1.9877x; 1.5994x over previous
import functools

import jax
import jax.numpy as jnp
from jax import lax
from jax.experimental import pallas as pl
from jax.experimental.pallas import tpu as pltpu
from jax.experimental.pallas import tpu_sc as plsc

N = 10000
E = 320000
D = 128

_NC = 2
_NS = 16
_NW = _NC * _NS
_CHUNK = 80
_PER_W = E // _NW
_T = _PER_W // _CHUNK


def _gather_mul_body(x_hbm, ea_pos, eb_pos, ea_neg, eb_neg, out_pos, out_neg,
                     idx, ra, rb, si, sga, sgb, swb):
    wid = lax.axis_index("s") * _NC + lax.axis_index("c")
    w_base = wid * _PER_W

    def run_set(ea, eb, out):
        def gstart(q):
            pltpu.async_copy(x_hbm.at[idx.at[q, 0]], ra.at[q], sga.at[q])
            pltpu.async_copy(x_hbm.at[idx.at[q, 1]], rb.at[q], sgb.at[q])

        def gwait(p):
            pltpu.make_async_copy(x_hbm.at[idx.at[p, 0]], ra.at[p],
                                  sga.at[p]).wait()
            pltpu.make_async_copy(x_hbm.at[idx.at[p, 1]], rb.at[p],
                                  sgb.at[p]).wait()

        def body(t, carry):
            p = lax.rem(t, 2)
            q = 1 - p
            base_t = w_base + t * _CHUNK
            base_n = base_t + _CHUNK

            @pl.when(t + 1 < _T)
            def _():
                pltpu.async_copy(ea.at[pl.ds(base_n, _CHUNK)],
                                 idx.at[q, 0], si.at[q])
                pltpu.async_copy(eb.at[pl.ds(base_n, _CHUNK)],
                                 idx.at[q, 1], si.at[q])

            gwait(p)

            def row_body(r, c):
                for k in range(D // 16):
                    sl = pl.ds(k * 16, 16)
                    ra[p, r, sl] = ra[p, r, sl] * rb[p, r, sl]
                return c

            lax.fori_loop(0, _CHUNK, row_body, 0, unroll=2)

            @pl.when(t + 1 < _T)
            def _():
                pltpu.make_async_copy(ea.at[pl.ds(base_n, _CHUNK)],
                                      idx.at[q, 0], si.at[q]).wait()
                pltpu.make_async_copy(eb.at[pl.ds(base_n, _CHUNK)],
                                      idx.at[q, 1], si.at[q]).wait()

                @pl.when(t >= 1)
                def _():
                    pltpu.make_async_copy(
                        ra.at[q], out.at[pl.ds(base_t - _CHUNK, _CHUNK)],
                        swb.at[q]).wait()

                gstart(q)

            pltpu.async_copy(ra.at[p], out.at[pl.ds(base_t, _CHUNK)],
                             swb.at[p])
            return carry

        pltpu.sync_copy(ea.at[pl.ds(w_base, _CHUNK)], idx.at[0, 0])
        pltpu.sync_copy(eb.at[pl.ds(w_base, _CHUNK)], idx.at[0, 1])
        gstart(0)
        lax.fori_loop(0, _T, body, 0)
        pltpu.make_async_copy(
            ra.at[0], out.at[pl.ds(w_base + (_T - 1) * _CHUNK, _CHUNK)],
            swb.at[0]).wait()
        pltpu.make_async_copy(
            ra.at[1], out.at[pl.ds(w_base + (_T - 2) * _CHUNK, _CHUNK)],
            swb.at[1]).wait()

    run_set(ea_pos, eb_pos, out_pos)
    run_set(ea_neg, eb_neg, out_neg)


def _gather_mul(x, ei_pos, ei_neg):
    mesh = plsc.VectorSubcoreMesh(core_axis_name="c", subcore_axis_name="s")
    f = functools.partial(
        pl.kernel,
        mesh=mesh,
        out_type=[
            jax.ShapeDtypeStruct((E, D), jnp.float32),
            jax.ShapeDtypeStruct((E, D), jnp.float32),
        ],
        scratch_types=[
            pltpu.VMEM((2, 2, _CHUNK), jnp.int32),
            pltpu.VMEM((2, _CHUNK, D), jnp.float32),
            pltpu.VMEM((2, _CHUNK, D), jnp.float32),
            pltpu.SemaphoreType.DMA((2,)),
            pltpu.SemaphoreType.DMA((2,)),
            pltpu.SemaphoreType.DMA((2,)),
            pltpu.SemaphoreType.DMA((2,)),
        ],
    )(_gather_mul_body)
    return f(x, ei_pos[0], ei_pos[1], ei_neg[0], ei_neg[1])


_B = 3200
_G = E // _B


def _decode_body(ep_ref, en_ref, wcat_ref, b1_ref, be1_ref, w2t_ref, b2_ref,
                 we2t_ref, be2_ref, attr_ref, pos_ref, neg_ref):
    wcat = wcat_ref[...]
    we2t = we2t_ref[...]
    be2 = be2_ref[...]
    h = jnp.maximum(ep_ref[...], 0.0)
    a = jnp.dot(h, wcat)
    a1 = jnp.maximum(a[:, :D] + b1_ref[...], 0.0)
    attr_ref[...] = jax.nn.sigmoid(jnp.dot(a1, w2t_ref[...]) + b2_ref[...])
    ae = jnp.maximum(a[:, D:] + be1_ref[...], 0.0)
    pos_ref[...] = jax.nn.sigmoid(jnp.dot(ae, we2t) + be2)
    hn = jnp.maximum(en_ref[...], 0.0)
    an = jnp.maximum(jnp.dot(hn, wcat[:, D:]) + be1_ref[...], 0.0)
    neg_ref[...] = jax.nn.sigmoid(jnp.dot(an, we2t) + be2)


def _decode(em_pos, em_neg, wcat_t, b1r, be1r, w2t8, b2r, we2t8, be2r):
    return pl.pallas_call(
        _decode_body,
        grid=(_G,),
        in_specs=[
            pl.BlockSpec((_B, D), lambda i: (i, 0)),
            pl.BlockSpec((_B, D), lambda i: (i, 0)),
            pl.BlockSpec((D, 2 * D), lambda i: (0, 0)),
            pl.BlockSpec((1, D), lambda i: (0, 0)),
            pl.BlockSpec((1, D), lambda i: (0, 0)),
            pl.BlockSpec((D, 8), lambda i: (0, 0)),
            pl.BlockSpec((1, 8), lambda i: (0, 0)),
            pl.BlockSpec((D, 8), lambda i: (0, 0)),
            pl.BlockSpec((1, 1), lambda i: (0, 0)),
        ],
        out_specs=[
            pl.BlockSpec((_B, 8), lambda i: (i, 0)),
            pl.BlockSpec((_B, 8), lambda i: (i, 0)),
            pl.BlockSpec((_B, 8), lambda i: (i, 0)),
        ],
        out_shape=[
            jax.ShapeDtypeStruct((E, 8), jnp.float32),
            jax.ShapeDtypeStruct((E, 8), jnp.float32),
            jax.ShapeDtypeStruct((E, 8), jnp.float32),
        ],
        compiler_params=pltpu.CompilerParams(
            dimension_semantics=("arbitrary",),
        ),
    )(em_pos, em_neg, wcat_t, b1r, be1r, w2t8, b2r, we2t8, be2r)


def kernel(x, edge_index, edge_index_neg, W1, b1, W2, b2, We1, be1, We2, be2):
    em_pos, em_neg = _gather_mul(x, edge_index, edge_index_neg)

    wcat_t = jnp.concatenate([W1, We1], axis=0).T
    w2t8 = jnp.pad(W2, ((0, 1), (0, 0))).T
    b2r = jnp.pad(b2, (0, 1)).reshape(1, 8)
    we2t8 = jnp.pad(We2, ((0, 7), (0, 0))).T
    attr8, pos8, neg8 = _decode(
        em_pos, em_neg, wcat_t, b1.reshape(1, D), be1.reshape(1, D),
        w2t8, b2r, we2t8, be2.reshape(1, 1))
    return attr8[:, :7], pos8[:, 0], neg8[:, 0]


# speedup vs baseline: 2.5741x; 1.2950x over previous
import functools

import jax
import jax.numpy as jnp
from jax import lax
from jax.experimental import pallas as pl
from jax.experimental.pallas import tpu as pltpu
from jax.experimental.pallas import tpu_sc as plsc

N = 10000
E = 320000
D = 128

_NC = 2
_NS = 16
_NW = _NC * _NS
_CHUNK = 80
_PER_W = E // _NW
_T = _PER_W // _CHUNK


_NBUF = 4


def _gather_mul_body(x_hbm, ec_pos, ec_neg, out_pos, out_neg,
                     idx, ra, rb, si, sga, sgb, swb):
    wid = lax.axis_index("s") * _NC + lax.axis_index("c")
    w_base = wid * _PER_W

    def run_set(ec, out):
        def istart(t, b):
            base = w_base + t * _CHUNK
            pltpu.async_copy(ec.at[pl.ds(base, _CHUNK)], idx.at[b, 0],
                             si.at[b])
            pltpu.async_copy(ec.at[pl.ds(E + base, _CHUNK)], idx.at[b, 1],
                             si.at[b])

        def iwait(t, b):
            base = w_base + t * _CHUNK
            pltpu.make_async_copy(ec.at[pl.ds(base, _CHUNK)], idx.at[b, 0],
                                  si.at[b]).wait()
            pltpu.make_async_copy(ec.at[pl.ds(E + base, _CHUNK)],
                                  idx.at[b, 1], si.at[b]).wait()

        def gstart(b):
            pltpu.async_copy(x_hbm.at[idx.at[b, 0]], ra.at[b], sga.at[b])
            pltpu.async_copy(x_hbm.at[idx.at[b, 1]], rb.at[b], sgb.at[b])

        def gwait(b):
            pltpu.make_async_copy(x_hbm.at[idx.at[b, 0]], ra.at[b],
                                  sga.at[b]).wait()
            pltpu.make_async_copy(x_hbm.at[idx.at[b, 1]], rb.at[b],
                                  sgb.at[b]).wait()

        def wbwait(t, b):
            pltpu.make_async_copy(
                ra.at[b], out.at[pl.ds(w_base + t * _CHUNK, _CHUNK)],
                swb.at[b]).wait()

        def body(t, carry):
            b0 = lax.rem(t, _NBUF)
            b2 = lax.rem(t + 2, _NBUF)
            b3 = lax.rem(t + 3, _NBUF)

            @pl.when(t + 3 < _T)
            def _():
                istart(t + 3, b3)

            @pl.when(t + 2 < _T)
            def _():
                iwait(t + 2, b2)

                @pl.when(t >= 2)
                def _():
                    wbwait(t - 2, b2)

                gstart(b2)

            gwait(b0)

            def row_body(r, c):
                for k in range(D // 16):
                    sl = pl.ds(k * 16, 16)
                    ra[b0, r, sl] = ra[b0, r, sl] * rb[b0, r, sl]
                return c

            lax.fori_loop(0, _CHUNK, row_body, 0, unroll=4)

            pltpu.async_copy(ra.at[b0],
                             out.at[pl.ds(w_base + t * _CHUNK, _CHUNK)],
                             swb.at[b0])
            return carry

        for t in range(3):
            istart(t, t)
        for t in range(2):
            iwait(t, t)
            gstart(t)
        lax.fori_loop(0, _T, body, 0)
        for t in range(_T - _NBUF, _T):
            wbwait(t, t % _NBUF)

    run_set(ec_pos, out_pos)
    run_set(ec_neg, out_neg)


def _gather_mul(x, ei_pos, ei_neg):
    mesh = plsc.VectorSubcoreMesh(core_axis_name="c", subcore_axis_name="s")
    f = functools.partial(
        pl.kernel,
        mesh=mesh,
        out_type=[
            jax.ShapeDtypeStruct((E, D), jnp.float32),
            jax.ShapeDtypeStruct((E, D), jnp.float32),
        ],
        scratch_types=[
            pltpu.VMEM((_NBUF, 2, _CHUNK), jnp.int32),
            pltpu.VMEM((_NBUF, _CHUNK, D), jnp.float32),
            pltpu.VMEM((_NBUF, _CHUNK, D), jnp.float32),
            pltpu.SemaphoreType.DMA((_NBUF,)),
            pltpu.SemaphoreType.DMA((_NBUF,)),
            pltpu.SemaphoreType.DMA((_NBUF,)),
            pltpu.SemaphoreType.DMA((_NBUF,)),
        ],
    )(_gather_mul_body)
    return f(x, ei_pos.reshape(2 * E), ei_neg.reshape(2 * E))


_B = 3200
_G = E // _B


def _decode_body(ep_ref, en_ref, wcat_ref, b1_ref, be1_ref, w2t_ref, b2_ref,
                 we2t_ref, be2_ref, attr_ref, pos_ref, neg_ref):
    wcat = wcat_ref[...]
    we2t = we2t_ref[...]
    be2 = be2_ref[...]
    h = jnp.maximum(ep_ref[...], 0.0)
    a = jnp.dot(h, wcat)
    a1 = jnp.maximum(a[:, :D] + b1_ref[...], 0.0)
    attr_ref[...] = jax.nn.sigmoid(jnp.dot(a1, w2t_ref[...]) + b2_ref[...])
    ae = jnp.maximum(a[:, D:] + be1_ref[...], 0.0)
    pos_ref[...] = jax.nn.sigmoid(jnp.dot(ae, we2t) + be2)
    hn = jnp.maximum(en_ref[...], 0.0)
    an = jnp.maximum(jnp.dot(hn, wcat[:, D:]) + be1_ref[...], 0.0)
    neg_ref[...] = jax.nn.sigmoid(jnp.dot(an, we2t) + be2)


def _decode(em_pos, em_neg, wcat_t, b1r, be1r, w2t8, b2r, we2t8, be2r):
    return pl.pallas_call(
        _decode_body,
        grid=(_G,),
        in_specs=[
            pl.BlockSpec((_B, D), lambda i: (i, 0)),
            pl.BlockSpec((_B, D), lambda i: (i, 0)),
            pl.BlockSpec((D, 2 * D), lambda i: (0, 0)),
            pl.BlockSpec((1, D), lambda i: (0, 0)),
            pl.BlockSpec((1, D), lambda i: (0, 0)),
            pl.BlockSpec((D, 8), lambda i: (0, 0)),
            pl.BlockSpec((1, 8), lambda i: (0, 0)),
            pl.BlockSpec((D, 8), lambda i: (0, 0)),
            pl.BlockSpec((1, 1), lambda i: (0, 0)),
        ],
        out_specs=[
            pl.BlockSpec((_B, 8), lambda i: (i, 0)),
            pl.BlockSpec((_B, 8), lambda i: (i, 0)),
            pl.BlockSpec((_B, 8), lambda i: (i, 0)),
        ],
        out_shape=[
            jax.ShapeDtypeStruct((E, 8), jnp.float32),
            jax.ShapeDtypeStruct((E, 8), jnp.float32),
            jax.ShapeDtypeStruct((E, 8), jnp.float32),
        ],
        compiler_params=pltpu.CompilerParams(
            dimension_semantics=("arbitrary",),
        ),
    )(em_pos, em_neg, wcat_t, b1r, be1r, w2t8, b2r, we2t8, be2r)


def kernel(x, edge_index, edge_index_neg, W1, b1, W2, b2, We1, be1, We2, be2):
    em_pos, em_neg = _gather_mul(x, edge_index, edge_index_neg)

    wcat_t = jnp.concatenate([W1, We1], axis=0).T
    w2t8 = jnp.pad(W2, ((0, 1), (0, 0))).T
    b2r = jnp.pad(b2, (0, 1)).reshape(1, 8)
    we2t8 = jnp.pad(We2, ((0, 7), (0, 0))).T
    attr8, pos8, neg8 = _decode(
        em_pos, em_neg, wcat_t, b1.reshape(1, D), be1.reshape(1, D),
        w2t8, b2r, we2t8, be2.reshape(1, 1))
    return attr8[:, :7], pos8[:, 0], neg8[:, 0]
